# baseline (speedup 1.0000x reference)
import jax
import jax.numpy as jnp
from jax import lax
from jax.experimental import pallas as pl
from jax.experimental.pallas import tpu as pltpu
from jax.experimental.pallas import tpu_sc as plsc

N = 4194304
GRID1 = 16
BLK1 = N // GRID1

NC = 2
NS = 16
NW = NC * NS
PW = N // NW
CHUNK = 8192
NCHUNK = PW // CHUNK
UNROLL = 4

NBITS = 15
B = 1 << NBITS
HR = 256
SHIFT = 32 - NBITS - 1

MIN_NEG = 41943


def _k1_bce(x_ref, t_ref, nl_ref, sp_ref, np_ref):
    i = pl.program_id(0)
    x = x_ref[...].reshape(BLK1 // 128, 128)
    t = t_ref[...].reshape(BLK1 // 128, 128)
    loss = jnp.maximum(x, 0.0) - x * t + jnp.log1p(jnp.exp(-jnp.abs(x)))
    nl = jnp.where(t == 0.0, loss, 0.0).astype(jnp.bfloat16)
    nl_ref[...] = nl.reshape(BLK1)
    ps = jnp.sum(loss * t)
    npos = jnp.sum(t)

    @pl.when(i == 0)
    def _():
        sp_ref[0, 0] = ps
        np_ref[0, 0] = npos

    @pl.when(i > 0)
    def _():
        sp_ref[0, 0] += ps
        np_ref[0, 0] += npos


def _sc_hist_body(nl_hbm, ocnt_hbm, osum_hbm, buf0, buf1, hcnt, hsum,
                  sem0, sem1):
    c = lax.axis_index("c")
    s = lax.axis_index("s")
    wid = s * NC + c
    base = wid * PW

    zeros16 = jnp.zeros((16,), jnp.float32)
    ones16 = jnp.ones((16,), jnp.float32)

    @plsc.parallel_loop(0, HR, unroll=4)
    def zbody(j):
        for u in range(8):
            hcnt[j, pl.ds(u * 16, 16)] = zeros16
            hsum[j, pl.ds(u * 16, 16)] = zeros16

    bufs = (buf0, buf1)
    sems = (sem0, sem1)

    def start(ci):
        return pltpu.async_copy(
            nl_hbm.at[pl.ds(base + ci * CHUNK, CHUNK)],
            bufs[ci % 2], sems[ci % 2])

    def process(buf):
        @plsc.parallel_loop(0, CHUNK // 32, unroll=UNROLL)
        def ibody(i):
            v32 = buf[pl.ds(i * 32, 32)]
            for v in plsc.unpack(v32, format=plsc.PackFormat.INTERLEAVED):
                m = v > 0.0
                bits = plsc.bitcast(v, jnp.int32)
                idx = lax.shift_right_logical(bits, SHIFT)
                hi = lax.shift_right_logical(idx, 7)
                lo = idx & 127
                plsc.addupdate_scatter(hcnt, [hi, lo], ones16, mask=m)
                plsc.addupdate_scatter(hsum, [hi, lo], v, mask=m)

    copies = [None, None]
    copies[0] = start(0)
    for ci in range(NCHUNK):
        if ci + 1 < NCHUNK:
            copies[(ci + 1) % 2] = start(ci + 1)
        copies[ci % 2].wait()
        process(bufs[ci % 2])

    pltpu.sync_copy(hcnt, ocnt_hbm.at[wid])
    pltpu.sync_copy(hsum, osum_hbm.at[wid])


def _sc_hist(nl_flat):
    mesh = plsc.VectorSubcoreMesh(core_axis_name="c", subcore_axis_name="s")
    f = pl.kernel(
        _sc_hist_body,
        out_type=[
            jax.ShapeDtypeStruct((NW, HR, 128), jnp.float32),
            jax.ShapeDtypeStruct((NW, HR, 128), jnp.float32),
        ],
        mesh=mesh,
        compiler_params=pltpu.CompilerParams(needs_layout_passes=False),
        scratch_types=[
            pltpu.VMEM((CHUNK,), jnp.bfloat16),
            pltpu.VMEM((CHUNK,), jnp.bfloat16),
            pltpu.VMEM((HR, 128), jnp.float32),
            pltpu.VMEM((HR, 128), jnp.float32),
            pltpu.SemaphoreType.DMA,
            pltpu.SemaphoreType.DMA,
        ],
    )
    return f(nl_flat)


def _k3_select(cnt_ref, sm_ref, sp_ref, np_ref, out_ref):
    npos = np_ref[0, 0]
    cnt = jnp.sum(cnt_ref[...], axis=0)
    sm = jnp.sum(sm_ref[...], axis=0)

    col = lax.broadcasted_iota(jnp.int32, (128, 128), 0)
    row = lax.broadcasted_iota(jnp.int32, (128, 128), 1)
    upper = (col <= row).astype(jnp.float32)
    incl_row = lax.dot(cnt, upper, precision=lax.Precision.HIGHEST,
                       preferred_element_type=jnp.float32)
    row_tot = incl_row[:, 127:128]
    i2 = lax.broadcasted_iota(jnp.int32, (HR, HR), 0)
    j2 = lax.broadcasted_iota(jnp.int32, (HR, HR), 1)
    lstrict = (j2 < i2).astype(jnp.float32)
    pref_rows = lax.dot(lstrict, row_tot, precision=lax.Precision.HIGHEST,
                        preferred_element_type=jnp.float32)
    incl = incl_row + pref_rows
    tot = jnp.sum(cnt)
    above = tot - incl

    npi = npos.astype(jnp.int32)
    maxneg = N - npi
    k = jnp.minimum(jnp.maximum(MIN_NEG, 5 * npi), maxneg)
    kf = k.astype(jnp.float32)

    w = jnp.clip((kf - above) / cnt, 0.0, 1.0)
    w = jnp.where(cnt > 0.0, w, 0.0)
    sum_neg = jnp.sum(sm * w)
    count = npos + kf
    out_ref[0, 0] = (sp_ref[0, 0] + sum_neg) / count


def kernel(input, target):
    nl, sp, npos = pl.pallas_call(
        _k1_bce,
        grid=(GRID1,),
        in_specs=[
            pl.BlockSpec((BLK1,), lambda i: (i,)),
            pl.BlockSpec((BLK1,), lambda i: (i,)),
        ],
        out_specs=[
            pl.BlockSpec((BLK1,), lambda i: (i,)),
            pl.BlockSpec(memory_space=pltpu.SMEM),
            pl.BlockSpec(memory_space=pltpu.SMEM),
        ],
        out_shape=[
            jax.ShapeDtypeStruct((N,), jnp.bfloat16),
            jax.ShapeDtypeStruct((1, 1), jnp.float32),
            jax.ShapeDtypeStruct((1, 1), jnp.float32),
        ],
    )(input, target)

    ocnt, osum = _sc_hist(nl)

    out = pl.pallas_call(
        _k3_select,
        in_specs=[
            pl.BlockSpec((NW, HR, 128), lambda: (0, 0, 0)),
            pl.BlockSpec((NW, HR, 128), lambda: (0, 0, 0)),
            pl.BlockSpec(memory_space=pltpu.SMEM),
            pl.BlockSpec(memory_space=pltpu.SMEM),
        ],
        out_specs=pl.BlockSpec(memory_space=pltpu.SMEM),
        out_shape=jax.ShapeDtypeStruct((1, 1), jnp.float32),
    )(ocnt, osum, sp, npos)

    return out[0, 0]


# speedup vs baseline: 1.0432x; 1.0432x over previous
import jax
import jax.numpy as jnp
from jax import lax
from jax.experimental import pallas as pl
from jax.experimental.pallas import tpu as pltpu
from jax.experimental.pallas import tpu_sc as plsc

N = 4194304
GRID1 = 8
BLK1 = N // GRID1

NC = 2
NS = 16
NW = NC * NS
PW = N // NW
CHUNK = 8192
NCHUNK = PW // CHUNK
UNROLL = 4

NBITS = 15
B = 1 << NBITS
HR = 256
SHIFT = 32 - NBITS - 1

MIN_NEG = 41943


def _k1_bce(x_ref, t_ref, nl_ref, sp_ref, np_ref):
    i = pl.program_id(0)
    x = x_ref[...].reshape(BLK1 // 128, 128)
    t = t_ref[...].reshape(BLK1 // 128, 128)
    loss = jnp.maximum(x, 0.0) - x * t + jnp.log1p(jnp.exp(-jnp.abs(x)))
    nl = jnp.where(t == 0.0, loss, 0.0).astype(jnp.bfloat16)
    nl_ref[...] = nl.reshape(BLK1)
    ps = jnp.sum(loss * t)
    npos = jnp.sum(t)

    @pl.when(i == 0)
    def _():
        sp_ref[0, 0] = ps
        np_ref[0, 0] = npos

    @pl.when(i > 0)
    def _():
        sp_ref[0, 0] += ps
        np_ref[0, 0] += npos


def _sc_hist_body(nl_hbm, ocnt_hbm, osum_hbm, buf0, buf1, hcnt, hsum,
                  sem0, sem1):
    c = lax.axis_index("c")
    s = lax.axis_index("s")
    wid = s * NC + c
    base = wid * PW

    zeros16 = jnp.zeros((16,), jnp.float32)
    ones16 = jnp.ones((16,), jnp.float32)

    @plsc.parallel_loop(0, HR, unroll=4)
    def zbody(j):
        for u in range(8):
            hcnt[j, pl.ds(u * 16, 16)] = zeros16
            hsum[j, pl.ds(u * 16, 16)] = zeros16

    bufs = (buf0, buf1)
    sems = (sem0, sem1)

    def start(ci):
        return pltpu.async_copy(
            nl_hbm.at[pl.ds(base + ci * CHUNK, CHUNK)],
            bufs[ci % 2], sems[ci % 2])

    def process(buf):
        @plsc.parallel_loop(0, CHUNK // 32, unroll=UNROLL)
        def ibody(i):
            v32 = buf[pl.ds(i * 32, 32)]
            for v in plsc.unpack(v32, format=plsc.PackFormat.INTERLEAVED):
                m = v > 0.0
                bits = plsc.bitcast(v, jnp.int32)
                idx = lax.shift_right_logical(bits, SHIFT)
                hi = lax.shift_right_logical(idx, 7)
                lo = idx & 127
                plsc.addupdate_scatter(hcnt, [hi, lo], ones16, mask=m)
                plsc.addupdate_scatter(hsum, [hi, lo], v, mask=m)

    copies = [None, None]
    copies[0] = start(0)
    for ci in range(NCHUNK):
        if ci + 1 < NCHUNK:
            copies[(ci + 1) % 2] = start(ci + 1)
        copies[ci % 2].wait()
        process(bufs[ci % 2])

    pltpu.sync_copy(hcnt, ocnt_hbm.at[wid])
    pltpu.sync_copy(hsum, osum_hbm.at[wid])


def _sc_hist(nl_flat):
    mesh = plsc.VectorSubcoreMesh(core_axis_name="c", subcore_axis_name="s")
    f = pl.kernel(
        _sc_hist_body,
        out_type=[
            jax.ShapeDtypeStruct((NW, HR, 128), jnp.float32),
            jax.ShapeDtypeStruct((NW, HR, 128), jnp.float32),
        ],
        mesh=mesh,
        compiler_params=pltpu.CompilerParams(needs_layout_passes=False),
        scratch_types=[
            pltpu.VMEM((CHUNK,), jnp.bfloat16),
            pltpu.VMEM((CHUNK,), jnp.bfloat16),
            pltpu.VMEM((HR, 128), jnp.float32),
            pltpu.VMEM((HR, 128), jnp.float32),
            pltpu.SemaphoreType.DMA,
            pltpu.SemaphoreType.DMA,
        ],
    )
    return f(nl_flat)


def _k3_select(cnt_ref, sm_ref, sp_ref, np_ref, out_ref):
    npos = np_ref[0, 0]
    cnt = jnp.sum(cnt_ref[...], axis=0)
    sm = jnp.sum(sm_ref[...], axis=0)

    col = lax.broadcasted_iota(jnp.int32, (128, 128), 0)
    row = lax.broadcasted_iota(jnp.int32, (128, 128), 1)
    upper = (col <= row).astype(jnp.float32)
    incl_row = lax.dot(cnt, upper, precision=lax.Precision.HIGHEST,
                       preferred_element_type=jnp.float32)
    row_tot = incl_row[:, 127:128]
    i2 = lax.broadcasted_iota(jnp.int32, (HR, HR), 0)
    j2 = lax.broadcasted_iota(jnp.int32, (HR, HR), 1)
    lstrict = (j2 < i2).astype(jnp.float32)
    pref_rows = lax.dot(lstrict, row_tot, precision=lax.Precision.HIGHEST,
                        preferred_element_type=jnp.float32)
    incl = incl_row + pref_rows
    tot = jnp.sum(cnt)
    above = tot - incl

    npi = npos.astype(jnp.int32)
    maxneg = N - npi
    k = jnp.minimum(jnp.maximum(MIN_NEG, 5 * npi), maxneg)
    kf = k.astype(jnp.float32)

    w = jnp.clip((kf - above) / cnt, 0.0, 1.0)
    w = jnp.where(cnt > 0.0, w, 0.0)
    sum_neg = jnp.sum(sm * w)
    count = npos + kf
    out_ref[0, 0] = (sp_ref[0, 0] + sum_neg) / count


def kernel(input, target):
    nl, sp, npos = pl.pallas_call(
        _k1_bce,
        grid=(GRID1,),
        in_specs=[
            pl.BlockSpec((BLK1,), lambda i: (i,)),
            pl.BlockSpec((BLK1,), lambda i: (i,)),
        ],
        out_specs=[
            pl.BlockSpec((BLK1,), lambda i: (i,)),
            pl.BlockSpec(memory_space=pltpu.SMEM),
            pl.BlockSpec(memory_space=pltpu.SMEM),
        ],
        out_shape=[
            jax.ShapeDtypeStruct((N,), jnp.bfloat16),
            jax.ShapeDtypeStruct((1, 1), jnp.float32),
            jax.ShapeDtypeStruct((1, 1), jnp.float32),
        ],
    )(input, target)

    ocnt, osum = _sc_hist(nl)

    out = pl.pallas_call(
        _k3_select,
        in_specs=[
            pl.BlockSpec((NW, HR, 128), lambda: (0, 0, 0)),
            pl.BlockSpec((NW, HR, 128), lambda: (0, 0, 0)),
            pl.BlockSpec(memory_space=pltpu.SMEM),
            pl.BlockSpec(memory_space=pltpu.SMEM),
        ],
        out_specs=pl.BlockSpec(memory_space=pltpu.SMEM),
        out_shape=jax.ShapeDtypeStruct((1, 1), jnp.float32),
    )(ocnt, osum, sp, npos)

    return out[0, 0]
